# baseline (speedup 1.0000x reference)
import jax
import jax.numpy as jnp
from jax.experimental import pallas as pl
from jax.experimental.pallas import tpu as pltpu


def _gcn_kernel(x_ref, adj_a_ref, adj_b_ref, w_ref, wr_ref, out_ref, support_ref):
    i = pl.program_id(0)

    @pl.when(i == 0)
    def _():
        support_ref[...] = jnp.dot(
            x_ref[...], w_ref[...], preferred_element_type=jnp.float32
        )

    bm = out_ref.shape[0]
    half = bm // 2
    x_blk = x_ref[pl.ds(i * bm, bm), :]
    root = jnp.dot(x_blk, wr_ref[...], preferred_element_type=jnp.float32)
    sup = support_ref[...]
    out_ref[0:half, :] = (
        jnp.dot(adj_a_ref[...], sup, preferred_element_type=jnp.float32)
        + root[0:half, :]
    )
    out_ref[half:bm, :] = (
        jnp.dot(adj_b_ref[...], sup, preferred_element_type=jnp.float32)
        + root[half:bm, :]
    )


def kernel(x, adj, weight, root_weight):
    n, d_in = x.shape
    d_out = weight.shape[1]
    bm = 400
    half = bm // 2
    return pl.pallas_call(
        _gcn_kernel,
        grid=(n // bm,),
        in_specs=[
            pl.BlockSpec((n, d_in), lambda i: (0, 0)),
            pl.BlockSpec((half, n), lambda i: (2 * i, 0)),
            pl.BlockSpec((half, n), lambda i: (2 * i + 1, 0)),
            pl.BlockSpec((d_in, d_out), lambda i: (0, 0)),
            pl.BlockSpec((d_in, d_out), lambda i: (0, 0)),
        ],
        out_specs=pl.BlockSpec((bm, d_out), lambda i: (i, 0)),
        out_shape=jax.ShapeDtypeStruct((n, d_out), jnp.float32),
        scratch_shapes=[pltpu.VMEM((n, d_out), jnp.float32)],
    )(x, adj, adj, weight, root_weight)


# speedup vs baseline: 1.0206x; 1.0206x over previous
import jax
import jax.numpy as jnp
from jax.experimental import pallas as pl
from jax.experimental.pallas import tpu as pltpu


def _gcn_kernel(x_ref, adj_ref, w_ref, wr_ref, out_ref, support_ref):
    i = pl.program_id(0)

    @pl.when(i == 0)
    def _():
        support_ref[...] = jnp.dot(
            x_ref[...], w_ref[...], preferred_element_type=jnp.float32
        )

    bm = out_ref.shape[0]
    x_blk = x_ref[pl.ds(i * bm, bm), :]
    acc = jnp.dot(adj_ref[...], support_ref[...], preferred_element_type=jnp.float32)
    acc = acc + jnp.dot(x_blk, wr_ref[...], preferred_element_type=jnp.float32)
    out_ref[...] = acc


def kernel(x, adj, weight, root_weight):
    n, d_in = x.shape
    d_out = weight.shape[1]
    bm = 400
    return pl.pallas_call(
        _gcn_kernel,
        grid=(n // bm,),
        in_specs=[
            pl.BlockSpec((n, d_in), lambda i: (0, 0)),
            pl.BlockSpec((bm, n), lambda i: (i, 0)),
            pl.BlockSpec((d_in, d_out), lambda i: (0, 0)),
            pl.BlockSpec((d_in, d_out), lambda i: (0, 0)),
        ],
        out_specs=pl.BlockSpec((bm, d_out), lambda i: (i, 0)),
        out_shape=jax.ShapeDtypeStruct((n, d_out), jnp.float32),
        scratch_shapes=[pltpu.VMEM((n, d_out), jnp.float32)],
    )(x, adj, weight, root_weight)
